# baseline (speedup 1.0000x reference)
import functools

import jax
import jax.numpy as jnp
import numpy as np
from jax import lax
from jax.experimental import pallas as pl
from jax.experimental.pallas import tpu as pltpu
from jax.experimental.pallas import tpu_sc as plsc

BS, SEQ, EMB = 4, 4096, 2048
TOTAL = BS * EMB
NC, NS = 2, 16
NW = NC * NS
PER_W = TOTAL // NW
CHUNK = 128
NCH = PER_W // CHUNK


def _rotl(x, d):
    return ((x << np.uint32(d)) | (x >> np.uint32(32 - d))).astype(np.uint32)


def _threefry2x32(k1, k2, c1, c2):
    rot_a = (13, 15, 26, 6)
    rot_b = (17, 29, 16, 24)
    ks = (np.uint32(k1), np.uint32(k2),
          np.uint32(k1) ^ np.uint32(k2) ^ np.uint32(0x1BD11BDA))
    x0 = (np.asarray(c1, np.uint32) + ks[0]).astype(np.uint32)
    x1 = (np.asarray(c2, np.uint32) + ks[1]).astype(np.uint32)
    for i in range(5):
        for r in (rot_a if i % 2 == 0 else rot_b):
            x0 = (x0 + x1).astype(np.uint32)
            x1 = _rotl(x1, r) ^ x0
        x0 = (x0 + ks[(i + 1) % 3]).astype(np.uint32)
        x1 = (x1 + ks[(i + 2) % 3] + np.uint32(i + 1)).astype(np.uint32)
    return x0, x1


def _sampled_indices(seed, n, span):
    b1, b2 = _threefry2x32(np.uint32(0), np.uint32(seed),
                           np.zeros(2, np.uint32),
                           np.arange(2, dtype=np.uint32))
    t1, t2 = _threefry2x32(b1[1], b2[1],
                           np.zeros(n, np.uint32),
                           np.arange(n, dtype=np.uint32))
    return ((t1 ^ t2) % np.uint32(span)).astype(np.int64)


_IDX = _sampled_indices(42, TOTAL, SEQ).reshape(BS, EMB)

_ROWS = (np.arange(BS, dtype=np.int64)[:, None] * SEQ + _IDX) \
    .astype(np.int32).reshape(TOTAL)


def _gather_call(x2, rows_flat):
    mesh = plsc.VectorSubcoreMesh(core_axis_name="c", subcore_axis_name="s")

    @functools.partial(
        pl.kernel,
        mesh=mesh,
        out_type=jax.ShapeDtypeStruct((BS, EMB), jnp.float32),
        scratch_types=[
            pltpu.VMEM((NCH, CHUNK), jnp.int32),
            pltpu.VMEM((NCH, CHUNK, CHUNK), jnp.float32),
            pltpu.VMEM((PER_W,), jnp.float32),
            pltpu.SemaphoreType.DMA,
        ],
        compiler_params=pltpu.CompilerParams(
            use_tc_tiling_on_sc=True, needs_layout_passes=False),
    )
    def body(x_hbm, rows_hbm, out_hbm, row_v, slab_v, val_v, sem):
        wid = lax.axis_index("s") * NC + lax.axis_index("c")
        base = wid * PER_W
        b = base // EMB
        j0 = base % EMB
        for c in range(NCH):
            pltpu.sync_copy(rows_hbm.at[pl.ds(base + c * CHUNK, CHUNK)],
                            row_v.at[c])
        copies = [
            pltpu.async_copy(
                x_hbm.at[row_v.at[c], pl.ds(j0 + c * CHUNK, CHUNK)],
                slab_v.at[c], sem)
            for c in range(NCH)
        ]
        for cp in copies:
            cp.wait()
        lane = lax.iota(jnp.int32, 16)
        for c in range(NCH):
            for k in range(CHUNK // 16):
                d = k * 16 + lane
                val_v[pl.ds(c * CHUNK + k * 16, 16)] = plsc.load_gather(
                    slab_v.at[c], [d, d])
        for c in range(NCH):
            pltpu.sync_copy(val_v.at[pl.ds(c * CHUNK, CHUNK)],
                            out_hbm.at[b, pl.ds(j0 + c * CHUNK, CHUNK)])

    return body(x2, rows_flat)


def kernel(output_tokens_from_bert):
    x = output_tokens_from_bert
    bs, seq, emb = x.shape
    return _gather_call(x.reshape(bs * seq, emb), jnp.asarray(_ROWS))


# speedup vs baseline: 1.0269x; 1.0269x over previous
import functools

import jax
import jax.numpy as jnp
import numpy as np
from jax import lax
from jax.experimental import pallas as pl
from jax.experimental.pallas import tpu as pltpu
from jax.experimental.pallas import tpu_sc as plsc

BS, SEQ, EMB = 4, 4096, 2048
TOTAL = BS * EMB
NC, NS = 2, 16
NW = NC * NS
PER_W = TOTAL // NW
CHUNK = 128
NCH = PER_W // CHUNK


def _rotl(x, d):
    return ((x << np.uint32(d)) | (x >> np.uint32(32 - d))).astype(np.uint32)


def _threefry2x32(k1, k2, c1, c2):
    rot_a = (13, 15, 26, 6)
    rot_b = (17, 29, 16, 24)
    ks = (np.uint32(k1), np.uint32(k2),
          np.uint32(k1) ^ np.uint32(k2) ^ np.uint32(0x1BD11BDA))
    x0 = (np.asarray(c1, np.uint32) + ks[0]).astype(np.uint32)
    x1 = (np.asarray(c2, np.uint32) + ks[1]).astype(np.uint32)
    for i in range(5):
        for r in (rot_a if i % 2 == 0 else rot_b):
            x0 = (x0 + x1).astype(np.uint32)
            x1 = _rotl(x1, r) ^ x0
        x0 = (x0 + ks[(i + 1) % 3]).astype(np.uint32)
        x1 = (x1 + ks[(i + 2) % 3] + np.uint32(i + 1)).astype(np.uint32)
    return x0, x1


def _sampled_indices(seed, n, span):
    b1, b2 = _threefry2x32(np.uint32(0), np.uint32(seed),
                           np.zeros(2, np.uint32),
                           np.arange(2, dtype=np.uint32))
    t1, t2 = _threefry2x32(b1[1], b2[1],
                           np.zeros(n, np.uint32),
                           np.arange(n, dtype=np.uint32))
    return ((t1 ^ t2) % np.uint32(span)).astype(np.int64)


_IDX = _sampled_indices(42, TOTAL, SEQ).reshape(BS, EMB)

_ROWS = (np.arange(BS, dtype=np.int64)[:, None] * SEQ + _IDX) \
    .astype(np.int32).reshape(NW * (PER_W // CHUNK), CHUNK)


def _gather_call(x2, rows_flat):
    mesh = plsc.VectorSubcoreMesh(core_axis_name="c", subcore_axis_name="s")

    @functools.partial(
        pl.kernel,
        mesh=mesh,
        out_type=jax.ShapeDtypeStruct((BS, EMB), jnp.float32),
        scratch_types=[
            pltpu.VMEM((NCH, CHUNK), jnp.int32),
            pltpu.VMEM((NCH, CHUNK, CHUNK), jnp.float32),
            pltpu.VMEM((PER_W,), jnp.float32),
            pltpu.SemaphoreType.DMA,
            pltpu.SemaphoreType.DMA,
            pltpu.SemaphoreType.DMA,
        ],
        compiler_params=pltpu.CompilerParams(
            use_tc_tiling_on_sc=True, needs_layout_passes=False),
    )
    def body(x_hbm, rows_hbm, out_hbm, row_v, slab_v, val_v,
             sem_i, sem_g, sem_o):
        wid = lax.axis_index("s") * NC + lax.axis_index("c")
        base = wid * PER_W
        b = base // EMB
        j0 = base % EMB
        lane = lax.iota(jnp.int32, 16)
        stages = [pltpu.async_copy(rows_hbm.at[pl.ds(wid * NCH + c, 1)],
                                   row_v.at[pl.ds(c, 1)], sem_i)
                  for c in range(NCH)]
        gathers = []
        for c in range(NCH):
            stages[c].wait()
            gathers.append(pltpu.async_copy(
                x_hbm.at[row_v.at[c], pl.ds(j0 + c * CHUNK, CHUNK)],
                slab_v.at[c], sem_g))
        writes = []
        for c in range(NCH):
            gathers[c].wait()
            for k in range(CHUNK // 16):
                d = k * 16 + lane
                val_v[pl.ds(c * CHUNK + k * 16, 16)] = plsc.load_gather(
                    slab_v.at[c], [d, d])
            writes.append(pltpu.async_copy(
                val_v.at[pl.ds(c * CHUNK, CHUNK)],
                out_hbm.at[b, pl.ds(j0 + c * CHUNK, CHUNK)], sem_o))
        for w in writes:
            w.wait()

    return body(x2, rows_flat)


def kernel(output_tokens_from_bert):
    x = output_tokens_from_bert
    bs, seq, emb = x.shape
    return _gather_call(x.reshape(bs * seq, emb), jnp.asarray(_ROWS))
